# baseline (speedup 1.0000x reference)
import functools

import jax
import jax.numpy as jnp
from jax import lax
from jax.experimental import pallas as pl
from jax.experimental.pallas import tpu as pltpu
from jax.experimental.pallas import tpu_sc as plsc

Q_LEN = 1
L = 32
BSZ = 64
NHID = 256
N = 32
DK = L * NHID
TOPK = 8
LANES = 16
NC = 2
NS = 16
NW = NC * NS
B_PER_W = BSZ // NW
SCALE = 1.0 / float(DK) ** 0.5


def _sc_body(q_hbm, keys_hbm, attn_hbm, topk_hbm, qv, kv, sv, tv, sem):
    wid = lax.axis_index("s") * NC + lax.axis_index("c")
    idx0 = lax.iota(jnp.int32, LANES)
    idx1 = idx0 + LANES

    def _round_bf16(x):
        u = plsc.bitcast(x, jnp.uint32)
        lsb = (u >> 16) & jnp.uint32(1)
        r = (u + jnp.uint32(0x7FFF) + lsb) & jnp.uint32(0xFFFF0000)
        return plsc.bitcast(r, jnp.float32)

    def _round_bf16_pair(x, y):
        return _round_bf16(x), _round_bf16(y)

    for rep in range(B_PER_W):
        b = wid * B_PER_W + rep
        for l in range(L):
            pltpu.sync_copy(q_hbm.at[0, l, b, :], qv.at[pl.ds(l * NHID, NHID)])

        def q_round_body(i, _):
            q0, q1 = _round_bf16_pair(qv[pl.ds(i * 2 * LANES, LANES)],
                                      qv[pl.ds(i * 2 * LANES + LANES, LANES)])
            qv[pl.ds(i * 2 * LANES, LANES)] = q0
            qv[pl.ds(i * 2 * LANES + LANES, LANES)] = q1
            return 0

        lax.fori_loop(0, DK // (2 * LANES), q_round_body, 0)

        s0 = jnp.zeros((LANES,), jnp.float32)
        s1 = jnp.zeros((LANES,), jnp.float32)
        for n in range(N):
            pltpu.sync_copy(keys_hbm.at[n, b, :], kv)

            def dot_body(i, acc):
                k0, k1 = _round_bf16_pair(kv[pl.ds(i * 2 * LANES, LANES)],
                                          kv[pl.ds(i * 2 * LANES + LANES, LANES)])
                return (acc + k0 * qv[pl.ds(i * 2 * LANES, LANES)]
                        + k1 * qv[pl.ds(i * 2 * LANES + LANES, LANES)])

            acc = lax.fori_loop(0, DK // (2 * LANES), dot_body,
                                jnp.zeros((LANES,), jnp.float32))
            score = jnp.sum(acc) * SCALE
            if n < LANES:
                s0 = jnp.where(idx0 == n, score, s0)
            else:
                s1 = jnp.where(idx0 == (n - LANES), score, s1)

        m = jnp.maximum(jnp.max(s0), jnp.max(s1))
        e0 = jnp.exp(s0 - m)
        e1 = jnp.exp(s1 - m)
        denom = jnp.sum(e0) + jnp.sum(e1)
        a0 = e0 / denom
        a1 = e1 / denom

        sv[pl.ds(0, LANES)] = a0
        sv[pl.ds(LANES, LANES)] = a1
        pltpu.sync_copy(sv, attn_hbm.at[pl.ds(b * N, N)])

        r0 = jnp.zeros((LANES,), jnp.int32)
        r1 = jnp.zeros((LANES,), jnp.int32)
        for mi in range(N):
            am_s = a0[mi] if mi < LANES else a1[mi - LANES]
            am = jnp.broadcast_to(am_s, (LANES,))
            r0 = r0 + (am > a0).astype(jnp.int32)
            r1 = r1 + (am > a1).astype(jnp.int32)
            r0 = r0 + ((am == a0) & (idx0 > mi)).astype(jnp.int32)
            r1 = r1 + ((am == a1) & (idx1 > mi)).astype(jnp.int32)

        plsc.store_scatter(tv, [r0], idx0, mask=r0 < TOPK)
        plsc.store_scatter(tv, [r1], idx1, mask=r1 < TOPK)
        pltpu.sync_copy(tv.at[pl.ds(0, TOPK)], topk_hbm.at[pl.ds(b * TOPK, TOPK)])


@functools.partial(
    pl.kernel,
    mesh=plsc.VectorSubcoreMesh(core_axis_name="c", subcore_axis_name="s"),
    out_type=[
        jax.ShapeDtypeStruct((BSZ * N,), jnp.float32),
        jax.ShapeDtypeStruct((BSZ * TOPK,), jnp.int32),
    ],
    scratch_types=[
        pltpu.VMEM((DK,), jnp.float32),
        pltpu.VMEM((DK,), jnp.float32),
        pltpu.VMEM((N,), jnp.float32),
        pltpu.VMEM((LANES,), jnp.int32),
        pltpu.SemaphoreType.DMA,
    ],
    compiler_params=pltpu.CompilerParams(needs_layout_passes=False),
)
def _sc_cache_attn(q_hbm, keys_hbm, attn_hbm, topk_hbm, qv, kv, sv, tv, sem):
    _sc_body(q_hbm, keys_hbm, attn_hbm, topk_hbm, qv, kv, sv, tv, sem)


def kernel(query, keys, values):
    del values
    attn_flat, topk_flat = _sc_cache_attn(query, keys)
    attention = attn_flat.reshape(BSZ, 1, N)
    topk_indices = topk_flat.reshape(BSZ, TOPK).T
    return attention, topk_indices


# speedup vs baseline: 2.2243x; 2.2243x over previous
import functools

import jax
import jax.numpy as jnp
from jax import lax
from jax.experimental import pallas as pl
from jax.experimental.pallas import tpu as pltpu
from jax.experimental.pallas import tpu_sc as plsc

Q_LEN = 1
L = 32
BSZ = 64
NHID = 256
N = 32
DK = L * NHID
TOPK = 8
LANES = 16
NC = 2
NS = 16
NW = NC * NS
B_PER_W = BSZ // NW
SCALE = 1.0 / float(DK) ** 0.5

SG = 4
DH = DK // 2
NGROUPS = N // SG


def _round_bf16_pair(x, y):
    pu = plsc.bitcast(plsc.pack(x, y, format=plsc.PackFormat.INTERLEAVED),
                      jnp.uint32)
    xr = plsc.bitcast(pu << 16, jnp.float32)
    yr = plsc.bitcast(pu & jnp.uint32(0xFFFF0000), jnp.float32)
    return xr, yr


def _sc_body(q_hbm, keys_hbm, attn_hbm, topk_hbm, qv, kb0, kb1, sv, tv,
             sem_q, sem0, sem1):
    wid = lax.axis_index("s") * NC + lax.axis_index("c")
    idx0 = lax.iota(jnp.int32, LANES)
    idx1 = idx0 + LANES
    kbufs = (kb0, kb1)
    sems = (sem0, sem1)

    for rep in range(B_PER_W):
        b = wid * B_PER_W + rep

        qcps = [pltpu.async_copy(q_hbm.at[0, l, b, :],
                                 qv.at[pl.ds(l * NHID, NHID)], sem_q)
                for l in range(L)]

        def fire(g):
            sg, h = divmod(g, 2)
            buf = kbufs[g % 2]
            sem = sems[g % 2]
            return [pltpu.async_copy(
                keys_hbm.at[sg * SG + s, b, pl.ds(h * DH, DH)],
                buf.at[s], sem) for s in range(SG)]

        cps = {0: fire(0)}

        for cp in qcps:
            cp.wait()

        def q_round_body(i, _):
            q0, q1 = _round_bf16_pair(qv[pl.ds(i * 2 * LANES, LANES)],
                                      qv[pl.ds(i * 2 * LANES + LANES, LANES)])
            qv[pl.ds(i * 2 * LANES, LANES)] = q0
            qv[pl.ds(i * 2 * LANES + LANES, LANES)] = q1
            return 0

        lax.fori_loop(0, DK // (2 * LANES), q_round_body, 0, unroll=4)

        s0 = jnp.zeros((LANES,), jnp.float32)
        s1 = jnp.zeros((LANES,), jnp.float32)
        accs = None
        for g in range(2 * NGROUPS):
            sg, h = divmod(g, 2)
            if g + 1 < 2 * NGROUPS:
                cps[g + 1] = fire(g + 1)
            for cp in cps.pop(g):
                cp.wait()
            buf = kbufs[g % 2]
            if h == 0:
                accs = (jnp.zeros((LANES,), jnp.float32),) * SG
            qoff = h * DH

            def dot_body(i, accs):
                q0 = qv[pl.ds(qoff + i * 2 * LANES, LANES)]
                q1 = qv[pl.ds(qoff + i * 2 * LANES + LANES, LANES)]
                out = []
                for s in range(SG):
                    k0, k1 = _round_bf16_pair(
                        buf[s, pl.ds(i * 2 * LANES, LANES)],
                        buf[s, pl.ds(i * 2 * LANES + LANES, LANES)])
                    out.append(accs[s] + k0 * q0 + k1 * q1)
                return tuple(out)

            accs = lax.fori_loop(0, DH // (2 * LANES), dot_body, accs,
                                 unroll=4)
            if h == 1:
                for s in range(SG):
                    n = sg * SG + s
                    score = jnp.sum(accs[s]) * SCALE
                    if n < LANES:
                        s0 = jnp.where(idx0 == n, score, s0)
                    else:
                        s1 = jnp.where(idx0 == (n - LANES), score, s1)

        m = jnp.maximum(jnp.max(s0), jnp.max(s1))
        e0 = jnp.exp(s0 - m)
        e1 = jnp.exp(s1 - m)
        denom = jnp.sum(e0) + jnp.sum(e1)
        a0 = e0 / denom
        a1 = e1 / denom

        sv[pl.ds(0, LANES)] = a0
        sv[pl.ds(LANES, LANES)] = a1
        pltpu.sync_copy(sv, attn_hbm.at[pl.ds(b * N, N)])

        r0 = jnp.zeros((LANES,), jnp.int32)
        r1 = jnp.zeros((LANES,), jnp.int32)
        for mi in range(N):
            am_s = a0[mi] if mi < LANES else a1[mi - LANES]
            am = jnp.broadcast_to(am_s, (LANES,))
            r0 = r0 + (am > a0).astype(jnp.int32)
            r1 = r1 + (am > a1).astype(jnp.int32)
            r0 = r0 + ((am == a0) & (idx0 > mi)).astype(jnp.int32)
            r1 = r1 + ((am == a1) & (idx1 > mi)).astype(jnp.int32)

        plsc.store_scatter(tv, [r0], idx0, mask=r0 < TOPK)
        plsc.store_scatter(tv, [r1], idx1, mask=r1 < TOPK)
        pltpu.sync_copy(tv.at[pl.ds(0, TOPK)], topk_hbm.at[pl.ds(b * TOPK, TOPK)])


@functools.partial(
    pl.kernel,
    mesh=plsc.VectorSubcoreMesh(core_axis_name="c", subcore_axis_name="s"),
    out_type=[
        jax.ShapeDtypeStruct((BSZ * N,), jnp.float32),
        jax.ShapeDtypeStruct((BSZ * TOPK,), jnp.int32),
    ],
    scratch_types=[
        pltpu.VMEM((DK,), jnp.float32),
        pltpu.VMEM((SG, DH), jnp.float32),
        pltpu.VMEM((SG, DH), jnp.float32),
        pltpu.VMEM((N,), jnp.float32),
        pltpu.VMEM((LANES,), jnp.int32),
        pltpu.SemaphoreType.DMA,
        pltpu.SemaphoreType.DMA,
        pltpu.SemaphoreType.DMA,
    ],
    compiler_params=pltpu.CompilerParams(needs_layout_passes=False),
)
def _sc_cache_attn(q_hbm, keys_hbm, attn_hbm, topk_hbm, qv, kb0, kb1, sv, tv,
                   sem_q, sem0, sem1):
    _sc_body(q_hbm, keys_hbm, attn_hbm, topk_hbm, qv, kb0, kb1, sv, tv,
             sem_q, sem0, sem1)


def kernel(query, keys, values):
    del values
    attn_flat, topk_flat = _sc_cache_attn(query, keys)
    attention = attn_flat.reshape(BSZ, 1, N)
    topk_indices = topk_flat.reshape(BSZ, TOPK).T
    return attention, topk_indices


# speedup vs baseline: 2.2704x; 1.0207x over previous
import functools

import jax
import jax.numpy as jnp
from jax import lax
from jax.experimental import pallas as pl
from jax.experimental.pallas import tpu as pltpu
from jax.experimental.pallas import tpu_sc as plsc

Q_LEN = 1
L = 32
BSZ = 64
NHID = 256
N = 32
DK = L * NHID
TOPK = 8
LANES = 16
NC = 2
NS = 16
NW = NC * NS
B_PER_W = BSZ // NW
SCALE = 1.0 / float(DK) ** 0.5

SG = 8
DH = 4096
NH = DK // DH
NGROUPS = N // SG


def _round_bf16_pair(x, y):
    pu = plsc.bitcast(plsc.pack(x, y, format=plsc.PackFormat.INTERLEAVED),
                      jnp.uint32)
    xr = plsc.bitcast(pu << 16, jnp.float32)
    yr = plsc.bitcast(pu & jnp.uint32(0xFFFF0000), jnp.float32)
    return xr, yr


def _sc_body(q_hbm, keys_hbm, attn_hbm, topk_hbm, qv, kb0, kb1, sv, tv,
             sem_q, sem0, sem1):
    wid = lax.axis_index("s") * NC + lax.axis_index("c")
    idx0 = lax.iota(jnp.int32, LANES)
    idx1 = idx0 + LANES
    kbufs = (kb0, kb1)
    sems = (sem0, sem1)

    for rep in range(B_PER_W):
        b = wid * B_PER_W + rep

        qcps = [pltpu.async_copy(q_hbm.at[0, l, b, :],
                                 qv.at[pl.ds(l * NHID, NHID)], sem_q)
                for l in range(L)]

        def fire(g):
            sg, h = divmod(g, NH)
            buf = kbufs[g % 2]
            sem = sems[g % 2]
            return [pltpu.async_copy(
                keys_hbm.at[sg * SG + s, b, pl.ds(h * DH, DH)],
                buf.at[s], sem) for s in range(SG)]

        cps = {0: fire(0)}

        for cp in qcps:
            cp.wait()

        @plsc.parallel_loop(0, DK // (2 * LANES), unroll=4)
        def q_round_body(i):
            q0, q1 = _round_bf16_pair(qv[pl.ds(i * 2 * LANES, LANES)],
                                      qv[pl.ds(i * 2 * LANES + LANES, LANES)])
            qv[pl.ds(i * 2 * LANES, LANES)] = q0
            qv[pl.ds(i * 2 * LANES + LANES, LANES)] = q1

        s0 = jnp.zeros((LANES,), jnp.float32)
        s1 = jnp.zeros((LANES,), jnp.float32)
        accs = None
        for g in range(NH * NGROUPS):
            sg, h = divmod(g, NH)
            if g + 1 < NH * NGROUPS:
                cps[g + 1] = fire(g + 1)
            for cp in cps.pop(g):
                cp.wait()
            buf = kbufs[g % 2]
            if h == 0:
                accs = (jnp.zeros((LANES,), jnp.float32),) * SG
            qoff = h * DH

            def dot_body(i, accs):
                q0 = qv[pl.ds(qoff + i * 2 * LANES, LANES)]
                q1 = qv[pl.ds(qoff + i * 2 * LANES + LANES, LANES)]
                out = []
                for s in range(SG):
                    k0, k1 = _round_bf16_pair(
                        buf[s, pl.ds(i * 2 * LANES, LANES)],
                        buf[s, pl.ds(i * 2 * LANES + LANES, LANES)])
                    out.append(accs[s] + k0 * q0 + k1 * q1)
                return tuple(out)

            accs = plsc.parallel_loop(0, DH // (2 * LANES), unroll=4,
                                      carry=accs)(dot_body)
            if h == NH - 1:
                for s in range(SG):
                    n = sg * SG + s
                    score = jnp.sum(accs[s]) * SCALE
                    if n < LANES:
                        s0 = jnp.where(idx0 == n, score, s0)
                    else:
                        s1 = jnp.where(idx0 == (n - LANES), score, s1)

        m = jnp.maximum(jnp.max(s0), jnp.max(s1))
        e0 = jnp.exp(s0 - m)
        e1 = jnp.exp(s1 - m)
        denom = jnp.sum(e0) + jnp.sum(e1)
        a0 = e0 / denom
        a1 = e1 / denom

        sv[pl.ds(0, LANES)] = a0
        sv[pl.ds(LANES, LANES)] = a1
        pltpu.sync_copy(sv, attn_hbm.at[pl.ds(b * N, N)])

        r0 = jnp.zeros((LANES,), jnp.int32)
        r1 = jnp.zeros((LANES,), jnp.int32)
        for mi in range(N):
            am_s = a0[mi] if mi < LANES else a1[mi - LANES]
            am = jnp.broadcast_to(am_s, (LANES,))
            r0 = r0 + (am > a0).astype(jnp.int32)
            r1 = r1 + (am > a1).astype(jnp.int32)
            r0 = r0 + ((am == a0) & (idx0 > mi)).astype(jnp.int32)
            r1 = r1 + ((am == a1) & (idx1 > mi)).astype(jnp.int32)

        plsc.store_scatter(tv, [r0], idx0, mask=r0 < TOPK)
        plsc.store_scatter(tv, [r1], idx1, mask=r1 < TOPK)
        pltpu.sync_copy(tv.at[pl.ds(0, TOPK)], topk_hbm.at[pl.ds(b * TOPK, TOPK)])


@functools.partial(
    pl.kernel,
    mesh=plsc.VectorSubcoreMesh(core_axis_name="c", subcore_axis_name="s"),
    out_type=[
        jax.ShapeDtypeStruct((BSZ * N,), jnp.float32),
        jax.ShapeDtypeStruct((BSZ * TOPK,), jnp.int32),
    ],
    scratch_types=[
        pltpu.VMEM((DK,), jnp.float32),
        pltpu.VMEM((SG, DH), jnp.float32),
        pltpu.VMEM((SG, DH), jnp.float32),
        pltpu.VMEM((N,), jnp.float32),
        pltpu.VMEM((LANES,), jnp.int32),
        pltpu.SemaphoreType.DMA,
        pltpu.SemaphoreType.DMA,
        pltpu.SemaphoreType.DMA,
    ],
    compiler_params=pltpu.CompilerParams(needs_layout_passes=False),
)
def _sc_cache_attn(q_hbm, keys_hbm, attn_hbm, topk_hbm, qv, kb0, kb1, sv, tv,
                   sem_q, sem0, sem1):
    _sc_body(q_hbm, keys_hbm, attn_hbm, topk_hbm, qv, kb0, kb1, sv, tv,
             sem_q, sem0, sem1)


def kernel(query, keys, values):
    del values
    attn_flat, topk_flat = _sc_cache_attn(query, keys)
    attention = attn_flat.reshape(BSZ, 1, N)
    topk_indices = topk_flat.reshape(BSZ, TOPK).T
    return attention, topk_indices
